# baseline (speedup 1.0000x reference)
import functools

import jax
import jax.numpy as jnp
from jax import lax
from jax.experimental import pallas as pl
from jax.experimental.pallas import tpu as pltpu
from jax.experimental.pallas import tpu_sc as plsc

_M, _D = 16384, 128
_N = _M * _D // 2
_NC, _NS = 2, 16
_NW = _NC * _NS
_CHUNK = _N // _NW
_ROWS = _CHUNK * 2 // _D

_CSIZES = (16, 32, 32, 48, 48, 48, 24, 8)
_NCHUNK = len(_CSIZES)

_mesh = plsc.VectorSubcoreMesh(core_axis_name="c", subcore_axis_name="s")


@functools.partial(
    pl.kernel,
    out_type=jax.ShapeDtypeStruct((_M, _D), jnp.float32),
    mesh=_mesh,
    scratch_types=[
        pltpu.VMEM((_CHUNK // _D, _D), jnp.float32),
        pltpu.VMEM((_ROWS, _D), jnp.float32),
        pltpu.SemaphoreType.DMA((_NCHUNK,)),
        pltpu.SemaphoreType.DMA,
    ],
    compiler_params=pltpu.CompilerParams(needs_layout_passes=False,
                                         disable_bounds_checks=True),
)
def _extend(x_hbm, out_hbm, xbuf, buf, insem, outsem):
    wid = lax.axis_index("s") * _NC + lax.axis_index("c")
    nrows = _CHUNK // _D
    starts = [sum(_CSIZES[:c]) for c in range(_NCHUNK)]
    xrow0 = wid * nrows
    row0 = wid * _ROWS

    in_copies = [
        pltpu.make_async_copy(
            x_hbm.at[pl.ds(xrow0 + starts[c], _CSIZES[c]), :],
            xbuf.at[pl.ds(starts[c], _CSIZES[c]), :],
            insem.at[c],
        )
        for c in range(_NCHUNK)
    ]
    for cp in in_copies:
        cp.start()

    lanes = lax.iota(jnp.int32, 16)
    odd_cols = [lanes * 2 + 1 + 32 * t for t in range(4)]
    even_cols = [lanes * 2 + 32 * t for t in range(4)]
    nan16 = jnp.full((16,), jnp.nan, jnp.float32)
    one16 = jnp.full((16,), 1, jnp.int32)

    def body(i, row):
        rows = [row, row + one16]
        vals = [xbuf[i, pl.ds(16 * t, 16)] for t in range(8)]
        for t in range(8):
            plsc.store_scatter(buf, [rows[t // 4], odd_cols[t % 4]], vals[t])
        for p in range(2):
            for t in range(4):
                plsc.store_scatter(buf, [rows[p], even_cols[t]], nan16)
        return row + 2 * one16

    out_copies = []
    for c in range(_NCHUNK):
        in_copies[c].wait()
        lax.fori_loop(starts[c], starts[c] + _CSIZES[c], body,
                      jnp.full((16,), 2 * starts[c], jnp.int32))
        cp = pltpu.make_async_copy(
            buf.at[pl.ds(2 * starts[c], 2 * _CSIZES[c]), :],
            out_hbm.at[pl.ds(row0 + 2 * starts[c], 2 * _CSIZES[c]), :],
            outsem,
        )
        cp.start()
        out_copies.append(cp)
    for cp in out_copies:
        cp.wait()


def kernel(x):
    return _extend(x)


# speedup vs baseline: 1.0165x; 1.0165x over previous
import functools

import jax
import jax.numpy as jnp
from jax import lax
from jax.experimental import pallas as pl
from jax.experimental.pallas import tpu as pltpu
from jax.experimental.pallas import tpu_sc as plsc

_M, _D = 16384, 128
_N = _M * _D // 2
_NC, _NS = 2, 16
_NW = _NC * _NS
_CHUNK = _N // _NW
_ROWS = _CHUNK * 2 // _D

_CSIZES = (16, 32, 32, 48, 48, 32, 32, 16)
_NCHUNK = len(_CSIZES)

_mesh = plsc.VectorSubcoreMesh(core_axis_name="c", subcore_axis_name="s")


@functools.partial(
    pl.kernel,
    out_type=jax.ShapeDtypeStruct((_M, _D), jnp.float32),
    mesh=_mesh,
    scratch_types=[
        pltpu.VMEM((_CHUNK // _D, _D), jnp.float32),
        pltpu.VMEM((_ROWS, _D), jnp.float32),
        pltpu.SemaphoreType.DMA((_NCHUNK,)),
        pltpu.SemaphoreType.DMA,
    ],
    compiler_params=pltpu.CompilerParams(needs_layout_passes=False,
                                         disable_bounds_checks=True),
)
def _extend(x_hbm, out_hbm, xbuf, buf, insem, outsem):
    wid = lax.axis_index("s") * _NC + lax.axis_index("c")
    nrows = _CHUNK // _D
    starts = [sum(_CSIZES[:c]) for c in range(_NCHUNK)]
    xrow0 = wid * nrows
    row0 = wid * _ROWS

    in_copies = [
        pltpu.make_async_copy(
            x_hbm.at[pl.ds(xrow0 + starts[c], _CSIZES[c]), :],
            xbuf.at[pl.ds(starts[c], _CSIZES[c]), :],
            insem.at[c],
        )
        for c in range(_NCHUNK)
    ]
    for cp in in_copies:
        cp.start()

    lanes = lax.iota(jnp.int32, 16)
    odd_cols = [lanes * 2 + 1 + 32 * t for t in range(4)]
    even_cols = [lanes * 2 + 32 * t for t in range(4)]
    nan16 = jnp.full((16,), jnp.nan, jnp.float32)
    one16 = jnp.full((16,), 1, jnp.int32)

    def body(i, row):
        rows = [row, row + one16]
        vals = [xbuf[i, pl.ds(16 * t, 16)] for t in range(8)]
        for t in range(8):
            plsc.store_scatter(buf, [rows[t // 4], odd_cols[t % 4]], vals[t])
        for p in range(2):
            for t in range(4):
                plsc.store_scatter(buf, [rows[p], even_cols[t]], nan16)
        return row + 2 * one16

    out_copies = []
    for c in range(_NCHUNK):
        in_copies[c].wait()
        lax.fori_loop(starts[c], starts[c] + _CSIZES[c], body,
                      jnp.full((16,), 2 * starts[c], jnp.int32))
        cp = pltpu.make_async_copy(
            buf.at[pl.ds(2 * starts[c], 2 * _CSIZES[c]), :],
            out_hbm.at[pl.ds(row0 + 2 * starts[c], 2 * _CSIZES[c]), :],
            outsem,
        )
        cp.start()
        out_copies.append(cp)
    for cp in out_copies:
        cp.wait()


def kernel(x):
    return _extend(x)
